# baseline (speedup 1.0000x reference)
import functools

import jax
import jax.numpy as jnp
from jax import lax
from jax.experimental import pallas as pl
from jax.experimental.pallas import tpu as pltpu
from jax.experimental.pallas import tpu_sc as plsc

F32 = jnp.float32
NC = 2
NS = 16
NW = NC * NS
LANES = 16
CHUNK = 128


def _wid():
    return lax.axis_index("c") * NS + lax.axis_index("s")


def _fill_rows(ref, rows, cols, value):
    npc = cols // LANES

    def body(t, carry):
        r = t // npc
        c = t % npc
        ref[r, pl.ds(c * LANES, LANES)] = jnp.full((LANES,), value, F32)
        return carry

    lax.fori_loop(0, rows * npc, body, 0)


def _make_deg_kernel(n_pad, n_edges):
    ep = n_edges // NW
    nch = ep // CHUNK
    tail = ep - nch * CHUNK
    rpt = n_pad // NS
    zc = 128
    mesh = plsc.VectorSubcoreMesh(core_axis_name="c", subcore_axis_name="s")

    @functools.partial(
        pl.kernel,
        out_type=jax.ShapeDtypeStruct((NC, n_pad, LANES), F32),
        mesh=mesh,
        compiler_params=pltpu.CompilerParams(use_tc_tiling_on_sc=False),
        scratch_types=[
            pltpu.VMEM_SHARED((n_pad, LANES), F32),
            pltpu.VMEM((CHUNK, LANES), F32),
            pltpu.VMEM((zc, LANES), F32),
            pltpu.VMEM((CHUNK,), jnp.int32),
            pltpu.VMEM((CHUNK,), jnp.int32),
            pltpu.VMEM((tail,), jnp.int32) if tail else None,
            pltpu.SemaphoreType.DMA,
            pltpu.SemaphoreType.DMA,
            pltpu.SemaphoreType.DMA,
            pltpu.SemaphoreType.DMA,
        ],
    )
    def deg_kernel(dst_hbm, out_hbm, acc, ones_v, zbuf, didx0, didx1, tidx,
                   ssem0, ssem1, isem0, isem1):
        c = lax.axis_index("c")
        s = lax.axis_index("s")
        wid = _wid()
        _fill_rows(zbuf, zc, LANES, 0.0)
        for k in range(rpt // zc):
            pltpu.sync_copy(zbuf, acc.at[pl.ds(s * rpt + k * zc, zc)])
        _fill_rows(ones_v, CHUNK, LANES, 1.0)
        plsc.subcore_barrier()

        tb = wid * ep
        dbufs = ((didx0, ssem0, isem0), (didx1, ssem1, isem1))

        def istart(j, b):
            di, _, isem = dbufs[b]
            pltpu.async_copy(dst_hbm.at[pl.ds(tb + j * CHUNK, CHUNK)], di, isem)

        def iwait(j, b):
            di, _, isem = dbufs[b]
            pltpu.make_async_copy(dst_hbm.at[pl.ds(tb + j * CHUNK, CHUNK)], di,
                                  isem).wait()

        def sstart(b):
            di, ssem, _ = dbufs[b]
            pltpu.async_copy(ones_v, acc.at[di], ssem, add=True)

        def swait(b):
            di, ssem, _ = dbufs[b]
            pltpu.make_async_copy(ones_v, acc.at[di], ssem).wait()

        istart(0, 0)
        istart(1, 1)
        iwait(0, 0)
        sstart(0)

        def pair(i, carry):
            iwait(2 * i + 1, 1)
            sstart(1)
            swait(0)
            istart(2 * i + 2, 0)
            iwait(2 * i + 2, 0)
            sstart(0)
            swait(1)
            istart(2 * i + 3, 1)
            return carry

        lax.fori_loop(0, (nch - 2) // 2, pair, 0)
        iwait(nch - 1, 1)
        sstart(1)
        swait(0)
        swait(1)
        if tail:
            pltpu.sync_copy(dst_hbm.at[pl.ds(tb + nch * CHUNK, tail)], tidx)
            pltpu.sync_copy(ones_v.at[pl.ds(0, tail)], acc.at[tidx], add=True)
        plsc.subcore_barrier()

        for k in range(rpt // zc):
            r0 = s * rpt + k * zc
            pltpu.sync_copy(acc.at[pl.ds(r0, zc)], zbuf)
            pltpu.sync_copy(zbuf, out_hbm.at[c, pl.ds(r0, zc)])

    return deg_kernel


def _make_edge_kernel(n_pad, n_edges, dim):
    ep = n_edges // NW
    nch = ep // CHUNK
    tail = ep - nch * CHUNK
    rpt = n_pad // NS
    zc = 128
    mesh = plsc.VectorSubcoreMesh(core_axis_name="c", subcore_axis_name="s")

    @functools.partial(
        pl.kernel,
        out_type=jax.ShapeDtypeStruct((NC, n_pad, dim), F32),
        mesh=mesh,
        scratch_types=[
            pltpu.VMEM_SHARED((n_pad, dim), F32),
            pltpu.VMEM((CHUNK, dim), F32),
            pltpu.VMEM((CHUNK, dim), F32),
            pltpu.VMEM((CHUNK,), jnp.int32),
            pltpu.VMEM((CHUNK,), jnp.int32),
            pltpu.VMEM((CHUNK,), jnp.int32),
            pltpu.VMEM((CHUNK,), jnp.int32),
            pltpu.VMEM((tail,), jnp.int32) if tail else None,
            pltpu.VMEM((tail,), jnp.int32) if tail else None,
            pltpu.VMEM((tail, dim), F32) if tail else None,
            pltpu.SemaphoreType.DMA,
            pltpu.SemaphoreType.DMA,
            pltpu.SemaphoreType.DMA,
            pltpu.SemaphoreType.DMA,
            pltpu.SemaphoreType.DMA,
            pltpu.SemaphoreType.DMA,
        ],
    )
    def edge_kernel(y_hbm, src_hbm, dst_hbm, out_hbm, acc, rows0, rows1,
                    sidx0, sidx1, didx0, didx1, tsi, tdi, trows,
                    gsem0, gsem1, isem0, isem1, ssem0, ssem1):
        c = lax.axis_index("c")
        s = lax.axis_index("s")
        wid = _wid()
        _fill_rows(rows0, zc, dim, 0.0)
        for k in range(rpt // zc):
            pltpu.sync_copy(rows0, acc.at[pl.ds(s * rpt + k * zc, zc)])
        plsc.subcore_barrier()

        tb = wid * ep

        bufs = ((sidx0, didx0, rows0, gsem0, isem0),
                (sidx1, didx1, rows1, gsem1, isem1))

        def istart(j, b):
            si, di, _, _, isem = bufs[b]
            pltpu.async_copy(src_hbm.at[pl.ds(tb + j * CHUNK, CHUNK)], si, isem)
            pltpu.async_copy(dst_hbm.at[pl.ds(tb + j * CHUNK, CHUNK)], di, isem)

        def iwait(j, b):
            si, di, _, _, isem = bufs[b]
            pltpu.make_async_copy(src_hbm.at[pl.ds(tb + j * CHUNK, CHUNK)], si,
                                  isem).wait()
            pltpu.make_async_copy(dst_hbm.at[pl.ds(tb + j * CHUNK, CHUNK)], di,
                                  isem).wait()

        def gstart(b):
            si, _, rows, gsem, _ = bufs[b]
            pltpu.async_copy(y_hbm.at[si], rows, gsem)

        def gwait(b):
            si, _, rows, gsem, _ = bufs[b]
            pltpu.make_async_copy(y_hbm.at[si], rows, gsem).wait()

        def scat(b):
            _, di, rows, _, _ = bufs[b]
            pltpu.sync_copy(rows, acc.at[di], add=True)

        istart(0, 0)
        istart(1, 1)
        iwait(0, 0)
        gstart(0)

        def pair(i, carry):
            iwait(2 * i + 1, 1)
            gstart(1)
            gwait(0)
            scat(0)
            istart(2 * i + 2, 0)
            gwait(1)
            scat(1)
            iwait(2 * i + 2, 0)
            gstart(0)
            istart(2 * i + 3, 1)
            return carry

        lax.fori_loop(0, (nch - 2) // 2, pair, 0)
        iwait(nch - 1, 1)
        gstart(1)
        gwait(0)
        scat(0)
        gwait(1)
        scat(1)
        if tail:
            pltpu.sync_copy(src_hbm.at[pl.ds(tb + nch * CHUNK, tail)], tsi)
            pltpu.sync_copy(dst_hbm.at[pl.ds(tb + nch * CHUNK, tail)], tdi)
            pltpu.async_copy(y_hbm.at[tsi], trows, gsem0).wait()
            pltpu.sync_copy(trows, acc.at[tdi], add=True)
        plsc.subcore_barrier()

        for k in range(rpt // zc):
            r0 = s * rpt + k * zc
            pltpu.sync_copy(acc.at[pl.ds(r0, zc)], rows0)
            pltpu.sync_copy(rows0, out_hbm.at[c, pl.ds(r0, zc)])

    return edge_kernel


def _gru_body(up_ref, wih_ref, whh_ref, bih_ref, bhh_ref, gcnw_ref, xw_ref):
    seq, bn, indim = up_ref.shape
    hdim = whh_ref.shape[1]
    wih = wih_ref[...]
    whh = whh_ref[...]
    bih = bih_ref[...]
    bhh = bhh_ref[...]
    dn = (((1,), (1,)), ((), ()))
    h = jnp.zeros((bn, hdim), F32)
    for t in range(seq):
        x = up_ref[t]
        gi = lax.dot_general(x, wih, dn, preferred_element_type=F32) + bih
        gh = lax.dot_general(h, whh, dn, preferred_element_type=F32) + bhh
        i_r, i_z, i_n = gi[:, :hdim], gi[:, hdim:2 * hdim], gi[:, 2 * hdim:]
        h_r, h_z, h_n = gh[:, :hdim], gh[:, hdim:2 * hdim], gh[:, 2 * hdim:]
        r = jax.nn.sigmoid(i_r + h_r)
        z = jax.nn.sigmoid(i_z + h_z)
        n = jnp.tanh(i_n + r * h_n)
        h = (1.0 - z) * n + z * h
    xw_ref[...] = jnp.dot(h, gcnw_ref[...], preferred_element_type=F32)


def _run_gru(up, wih, whh, bih, bhh, gcnw, bn):
    seq, n_nodes, indim = up.shape
    gdim = gcnw.shape[1]
    grid = (n_nodes // bn,)
    return pl.pallas_call(
        _gru_body,
        grid=grid,
        in_specs=[
            pl.BlockSpec((seq, bn, indim), lambda i: (0, i, 0)),
            pl.BlockSpec(wih.shape, lambda i: (0, 0)),
            pl.BlockSpec(whh.shape, lambda i: (0, 0)),
            pl.BlockSpec(bih.shape, lambda i: (0, 0)),
            pl.BlockSpec(bhh.shape, lambda i: (0, 0)),
            pl.BlockSpec(gcnw.shape, lambda i: (0, 0)),
        ],
        out_specs=pl.BlockSpec((bn, gdim), lambda i: (i, 0)),
        out_shape=jax.ShapeDtypeStruct((n_nodes, gdim), F32),
    )(up, wih, whh, bih, bhh, gcnw)


def _scale_body(xw_ref, degp_ref, y_ref):
    deg = degp_ref[0, :, 0:1] + degp_ref[1, :, 0:1] + 1.0
    y_ref[...] = xw_ref[...] * lax.rsqrt(deg)


def _run_scale(xw, degp, bn):
    n_nodes, gdim = xw.shape
    return pl.pallas_call(
        _scale_body,
        grid=(n_nodes // bn,),
        in_specs=[
            pl.BlockSpec((bn, gdim), lambda i: (i, 0)),
            pl.BlockSpec((NC, bn, LANES), lambda i: (0, i, 0)),
        ],
        out_specs=pl.BlockSpec((bn, gdim), lambda i: (i, 0)),
        out_shape=jax.ShapeDtypeStruct((n_nodes, gdim), F32),
    )(xw, degp)


def _head_body(accp_ref, y_ref, degp_ref, gcnb_ref, f1w_ref, f1b_ref,
               f2w_ref, f2b_ref, clw_ref, clb_ref, md_ref, cl_ref):
    deg = degp_ref[0, :, 0:1] + degp_ref[1, :, 0:1] + 1.0
    dinv = lax.rsqrt(deg)
    t = (accp_ref[0] + accp_ref[1] + y_ref[...]) * dinv + gcnb_ref[...]
    h1 = jnp.maximum(
        jnp.dot(t, f1w_ref[...], preferred_element_type=F32) + f1b_ref[...],
        0.0)
    md_ref[...] = (jnp.dot(h1, f2w_ref[...], preferred_element_type=F32)
                   + f2b_ref[...])
    cl_ref[...] = (jnp.dot(t, clw_ref[...], preferred_element_type=F32)
                   + clb_ref[...])


def _run_heads(accp, y, degp, gcnb, f1w, f1b, f2w, f2b, clw, clb, bn):
    n_nodes, gdim = y.shape
    ff = f1w.shape[1]
    k = clw.shape[1]
    grid = (n_nodes // bn,)
    return pl.pallas_call(
        _head_body,
        grid=grid,
        in_specs=[
            pl.BlockSpec((NC, bn, gdim), lambda i: (0, i, 0)),
            pl.BlockSpec((bn, gdim), lambda i: (i, 0)),
            pl.BlockSpec((NC, bn, LANES), lambda i: (0, i, 0)),
            pl.BlockSpec(gcnb.shape, lambda i: (0, 0)),
            pl.BlockSpec(f1w.shape, lambda i: (0, 0)),
            pl.BlockSpec(f1b.shape, lambda i: (0, 0)),
            pl.BlockSpec(f2w.shape, lambda i: (0, 0)),
            pl.BlockSpec(f2b.shape, lambda i: (0, 0)),
            pl.BlockSpec(clw.shape, lambda i: (0, 0)),
            pl.BlockSpec(clb.shape, lambda i: (0, 0)),
        ],
        out_specs=[
            pl.BlockSpec((bn, 1), lambda i: (i, 0)),
            pl.BlockSpec((bn, k), lambda i: (i, 0)),
        ],
        out_shape=[
            jax.ShapeDtypeStruct((n_nodes, 1), F32),
            jax.ShapeDtypeStruct((n_nodes, k), F32),
        ],
    )(accp, y, degp, gcnb, f1w, f1b, f2w, f2b, clw, clb)


def kernel(user_profiles, interactions, edge_index, W_ih, W_hh, b_ih, b_hh,
           gcn_w, gcn_b, ff1_w, ff1_b, ff2_w, ff2_b, cl_w, cl_b):
    del interactions
    seq, n_nodes, _ = user_profiles.shape
    n_edges = edge_index.shape[1]
    gdim = gcn_w.shape[1]
    bn = 5000
    n_pad = -(-n_nodes // (NS * 128)) * (NS * 128)

    src = edge_index[0].astype(jnp.int32)
    dst = edge_index[1].astype(jnp.int32)

    degp = _make_deg_kernel(n_pad, n_edges)(dst)
    xw = _run_gru(user_profiles, W_ih, W_hh, b_ih.reshape(1, -1),
                  b_hh.reshape(1, -1), gcn_w, bn)
    y = _run_scale(xw, degp, bn)
    accp = _make_edge_kernel(n_pad, n_edges, gdim)(y, src, dst)
    max_depth, clusters = _run_heads(
        accp, y, degp, gcn_b.reshape(1, -1), ff1_w, ff1_b.reshape(1, -1),
        ff2_w, ff2_b.reshape(1, -1), cl_w, cl_b.reshape(1, -1), bn)
    return (max_depth, clusters)


# speedup vs baseline: 1.0276x; 1.0276x over previous
import functools

import jax
import jax.numpy as jnp
from jax import lax
from jax.experimental import pallas as pl
from jax.experimental.pallas import tpu as pltpu
from jax.experimental.pallas import tpu_sc as plsc

F32 = jnp.float32
NC = 2
NS = 16
NW = NC * NS
LANES = 16
CHUNK = 128


def _wid():
    return lax.axis_index("c") * NS + lax.axis_index("s")


def _fill_rows(ref, rows, cols, value):
    npc = cols // LANES

    def body(t, carry):
        r = t // npc
        c = t % npc
        ref[r, pl.ds(c * LANES, LANES)] = jnp.full((LANES,), value, F32)
        return carry

    lax.fori_loop(0, rows * npc, body, 0)


def _make_deg_kernel(n_pad, n_edges):
    ep = n_edges // NW
    nch = ep // CHUNK
    tail = ep - nch * CHUNK
    rpt = n_pad // NS
    zc = 128
    mesh = plsc.VectorSubcoreMesh(core_axis_name="c", subcore_axis_name="s")

    @functools.partial(
        pl.kernel,
        out_type=jax.ShapeDtypeStruct((NC, n_pad, LANES), F32),
        mesh=mesh,
        compiler_params=pltpu.CompilerParams(use_tc_tiling_on_sc=False),
        scratch_types=[
            pltpu.VMEM_SHARED((n_pad, LANES), F32),
            pltpu.VMEM((CHUNK, LANES), F32),
            pltpu.VMEM((zc, LANES), F32),
            pltpu.VMEM((CHUNK,), jnp.int32),
            pltpu.VMEM((CHUNK,), jnp.int32),
            pltpu.VMEM((tail,), jnp.int32) if tail else None,
            pltpu.SemaphoreType.DMA,
            pltpu.SemaphoreType.DMA,
            pltpu.SemaphoreType.DMA,
            pltpu.SemaphoreType.DMA,
        ],
    )
    def deg_kernel(dst_hbm, out_hbm, acc, ones_v, zbuf, didx0, didx1, tidx,
                   ssem0, ssem1, isem0, isem1):
        c = lax.axis_index("c")
        s = lax.axis_index("s")
        wid = _wid()
        _fill_rows(zbuf, zc, LANES, 0.0)
        for k in range(rpt // zc):
            pltpu.sync_copy(zbuf, acc.at[pl.ds(s * rpt + k * zc, zc)])
        _fill_rows(ones_v, CHUNK, LANES, 1.0)
        plsc.subcore_barrier()

        tb = wid * ep
        dbufs = ((didx0, ssem0, isem0), (didx1, ssem1, isem1))

        def istart(j, b):
            di, _, isem = dbufs[b]
            pltpu.async_copy(dst_hbm.at[pl.ds(tb + j * CHUNK, CHUNK)], di, isem)

        def iwait(j, b):
            di, _, isem = dbufs[b]
            pltpu.make_async_copy(dst_hbm.at[pl.ds(tb + j * CHUNK, CHUNK)], di,
                                  isem).wait()

        def sstart(b):
            di, ssem, _ = dbufs[b]
            pltpu.async_copy(ones_v, acc.at[di], ssem, add=True)

        def swait(b):
            di, ssem, _ = dbufs[b]
            pltpu.make_async_copy(ones_v, acc.at[di], ssem).wait()

        istart(0, 0)
        istart(1, 1)
        iwait(0, 0)
        sstart(0)

        def pair(i, carry):
            iwait(2 * i + 1, 1)
            sstart(1)
            swait(0)
            istart(2 * i + 2, 0)
            iwait(2 * i + 2, 0)
            sstart(0)
            swait(1)
            istart(2 * i + 3, 1)
            return carry

        lax.fori_loop(0, (nch - 2) // 2, pair, 0)
        iwait(nch - 1, 1)
        sstart(1)
        swait(0)
        swait(1)
        if tail:
            pltpu.sync_copy(dst_hbm.at[pl.ds(tb + nch * CHUNK, tail)], tidx)
            pltpu.sync_copy(ones_v.at[pl.ds(0, tail)], acc.at[tidx], add=True)
        plsc.subcore_barrier()

        for k in range(rpt // zc):
            r0 = s * rpt + k * zc
            pltpu.sync_copy(acc.at[pl.ds(r0, zc)], zbuf)
            pltpu.sync_copy(zbuf, out_hbm.at[c, pl.ds(r0, zc)])

    return deg_kernel


def _make_edge_kernel(n_pad, n_edges, dim):
    ep = n_edges // NW
    nch = ep // CHUNK
    tail = ep - nch * CHUNK
    rpt = n_pad // NS
    zc = 128
    mesh = plsc.VectorSubcoreMesh(core_axis_name="c", subcore_axis_name="s")

    @functools.partial(
        pl.kernel,
        out_type=jax.ShapeDtypeStruct((NC, n_pad, dim), F32),
        mesh=mesh,
        scratch_types=[
            pltpu.VMEM_SHARED((n_pad, dim), F32),
            pltpu.VMEM((CHUNK, dim), F32),
            pltpu.VMEM((CHUNK, dim), F32),
            pltpu.VMEM((CHUNK,), jnp.int32),
            pltpu.VMEM((CHUNK,), jnp.int32),
            pltpu.VMEM((CHUNK,), jnp.int32),
            pltpu.VMEM((CHUNK,), jnp.int32),
            pltpu.VMEM((tail,), jnp.int32) if tail else None,
            pltpu.VMEM((tail,), jnp.int32) if tail else None,
            pltpu.VMEM((tail, dim), F32) if tail else None,
            pltpu.SemaphoreType.DMA,
            pltpu.SemaphoreType.DMA,
            pltpu.SemaphoreType.DMA,
            pltpu.SemaphoreType.DMA,
            pltpu.SemaphoreType.DMA,
            pltpu.SemaphoreType.DMA,
        ],
    )
    def edge_kernel(y_hbm, src_hbm, dst_hbm, out_hbm, acc, rows0, rows1,
                    sidx0, sidx1, didx0, didx1, tsi, tdi, trows,
                    gsem0, gsem1, isem0, isem1, ssem0, ssem1):
        c = lax.axis_index("c")
        s = lax.axis_index("s")
        wid = _wid()
        _fill_rows(rows0, zc, dim, 0.0)
        for k in range(rpt // zc):
            pltpu.sync_copy(rows0, acc.at[pl.ds(s * rpt + k * zc, zc)])
        plsc.subcore_barrier()

        tb = wid * ep

        bufs = ((sidx0, didx0, rows0, gsem0, isem0),
                (sidx1, didx1, rows1, gsem1, isem1))

        def istart(j, b):
            si, di, _, _, isem = bufs[b]
            pltpu.async_copy(src_hbm.at[pl.ds(tb + j * CHUNK, CHUNK)], si, isem)
            pltpu.async_copy(dst_hbm.at[pl.ds(tb + j * CHUNK, CHUNK)], di, isem)

        def iwait(j, b):
            si, di, _, _, isem = bufs[b]
            pltpu.make_async_copy(src_hbm.at[pl.ds(tb + j * CHUNK, CHUNK)], si,
                                  isem).wait()
            pltpu.make_async_copy(dst_hbm.at[pl.ds(tb + j * CHUNK, CHUNK)], di,
                                  isem).wait()

        def gstart(b):
            si, _, rows, gsem, _ = bufs[b]
            pltpu.async_copy(y_hbm.at[si], rows, gsem)

        def gwait(b):
            si, _, rows, gsem, _ = bufs[b]
            pltpu.make_async_copy(y_hbm.at[si], rows, gsem).wait()

        def scat(b):
            _, di, rows, _, _ = bufs[b]
            pltpu.sync_copy(rows, acc.at[di], add=True)

        istart(0, 0)
        istart(1, 1)
        iwait(0, 0)
        gstart(0)

        def pair(i, carry):
            iwait(2 * i + 1, 1)
            gstart(1)
            gwait(0)
            scat(0)
            istart(2 * i + 2, 0)
            gwait(1)
            scat(1)
            iwait(2 * i + 2, 0)
            gstart(0)
            istart(2 * i + 3, 1)
            return carry

        lax.fori_loop(0, (nch - 2) // 2, pair, 0)
        iwait(nch - 1, 1)
        gstart(1)
        gwait(0)
        scat(0)
        gwait(1)
        scat(1)
        if tail:
            pltpu.sync_copy(src_hbm.at[pl.ds(tb + nch * CHUNK, tail)], tsi)
            pltpu.sync_copy(dst_hbm.at[pl.ds(tb + nch * CHUNK, tail)], tdi)
            pltpu.async_copy(y_hbm.at[tsi], trows, gsem0).wait()
            pltpu.sync_copy(trows, acc.at[tdi], add=True)
        plsc.subcore_barrier()

        for k in range(rpt // zc):
            r0 = s * rpt + k * zc
            pltpu.sync_copy(acc.at[pl.ds(r0, zc)], rows0)
            pltpu.sync_copy(rows0, out_hbm.at[c, pl.ds(r0, zc)])

    return edge_kernel


def _gru_body(up_ref, wih_ref, whh_ref, bih_ref, bhh_ref, gcnw_ref, xw_ref):
    seq, bn, indim = up_ref.shape
    hdim = whh_ref.shape[1]
    wih = wih_ref[...]
    whh = whh_ref[...]
    bih = bih_ref[...]
    bhh = bhh_ref[...]
    dn = (((1,), (1,)), ((), ()))
    h = jnp.zeros((bn, hdim), F32)
    for t in range(seq):
        x = up_ref[t]
        gi = lax.dot_general(x, wih, dn, preferred_element_type=F32) + bih
        gh = lax.dot_general(h, whh, dn, preferred_element_type=F32) + bhh
        i_r, i_z, i_n = gi[:, :hdim], gi[:, hdim:2 * hdim], gi[:, 2 * hdim:]
        h_r, h_z, h_n = gh[:, :hdim], gh[:, hdim:2 * hdim], gh[:, 2 * hdim:]
        r = jax.nn.sigmoid(i_r + h_r)
        z = jax.nn.sigmoid(i_z + h_z)
        n = jnp.tanh(i_n + r * h_n)
        h = (1.0 - z) * n + z * h
    xw_ref[...] = jnp.dot(h, gcnw_ref[...], preferred_element_type=F32)


def _run_gru(up, wih, whh, bih, bhh, gcnw, bn):
    seq, n_nodes, indim = up.shape
    gdim = gcnw.shape[1]
    grid = (n_nodes // bn,)
    return pl.pallas_call(
        _gru_body,
        grid=grid,
        in_specs=[
            pl.BlockSpec((seq, bn, indim), lambda i: (0, i, 0)),
            pl.BlockSpec(wih.shape, lambda i: (0, 0)),
            pl.BlockSpec(whh.shape, lambda i: (0, 0)),
            pl.BlockSpec(bih.shape, lambda i: (0, 0)),
            pl.BlockSpec(bhh.shape, lambda i: (0, 0)),
            pl.BlockSpec(gcnw.shape, lambda i: (0, 0)),
        ],
        out_specs=pl.BlockSpec((bn, gdim), lambda i: (i, 0)),
        out_shape=jax.ShapeDtypeStruct((n_nodes, gdim), F32),
    )(up, wih, whh, bih, bhh, gcnw)


def _scale_body(xw_ref, degp_ref, y_ref):
    deg = degp_ref[0, :, 0:1] + degp_ref[1, :, 0:1] + 1.0
    y_ref[...] = xw_ref[...] * lax.rsqrt(deg)


def _run_scale(xw, degp, bn):
    n_nodes, gdim = xw.shape
    return pl.pallas_call(
        _scale_body,
        grid=(n_nodes // bn,),
        in_specs=[
            pl.BlockSpec((bn, gdim), lambda i: (i, 0)),
            pl.BlockSpec((NC, bn, LANES), lambda i: (0, i, 0)),
        ],
        out_specs=pl.BlockSpec((bn, gdim), lambda i: (i, 0)),
        out_shape=jax.ShapeDtypeStruct((n_nodes, gdim), F32),
    )(xw, degp)


def _head_body(accp_ref, y_ref, degp_ref, gcnb_ref, f1w_ref, f1b_ref,
               f2w_ref, f2b_ref, clw_ref, clb_ref, md_ref, cl_ref):
    deg = degp_ref[0, :, 0:1] + degp_ref[1, :, 0:1] + 1.0
    dinv = lax.rsqrt(deg)
    t = (accp_ref[0] + accp_ref[1] + y_ref[...]) * dinv + gcnb_ref[...]
    h1 = jnp.maximum(
        jnp.dot(t, f1w_ref[...], preferred_element_type=F32) + f1b_ref[...],
        0.0)
    md_ref[...] = (jnp.dot(h1, f2w_ref[...], preferred_element_type=F32)
                   + f2b_ref[...])
    cl_ref[...] = (jnp.dot(t, clw_ref[...], preferred_element_type=F32)
                   + clb_ref[...])


def _run_heads(accp, y, degp, gcnb, f1w, f1b, f2w, f2b, clw, clb, bn):
    n_nodes, gdim = y.shape
    ff = f1w.shape[1]
    k = clw.shape[1]
    grid = (n_nodes // bn,)
    return pl.pallas_call(
        _head_body,
        grid=grid,
        in_specs=[
            pl.BlockSpec((NC, bn, gdim), lambda i: (0, i, 0)),
            pl.BlockSpec((bn, gdim), lambda i: (i, 0)),
            pl.BlockSpec((NC, bn, LANES), lambda i: (0, i, 0)),
            pl.BlockSpec(gcnb.shape, lambda i: (0, 0)),
            pl.BlockSpec(f1w.shape, lambda i: (0, 0)),
            pl.BlockSpec(f1b.shape, lambda i: (0, 0)),
            pl.BlockSpec(f2w.shape, lambda i: (0, 0)),
            pl.BlockSpec(f2b.shape, lambda i: (0, 0)),
            pl.BlockSpec(clw.shape, lambda i: (0, 0)),
            pl.BlockSpec(clb.shape, lambda i: (0, 0)),
        ],
        out_specs=[
            pl.BlockSpec((bn, 1), lambda i: (i, 0)),
            pl.BlockSpec((bn, k), lambda i: (i, 0)),
        ],
        out_shape=[
            jax.ShapeDtypeStruct((n_nodes, 1), F32),
            jax.ShapeDtypeStruct((n_nodes, k), F32),
        ],
    )(accp, y, degp, gcnb, f1w, f1b, f2w, f2b, clw, clb)


def kernel(user_profiles, interactions, edge_index, W_ih, W_hh, b_ih, b_hh,
           gcn_w, gcn_b, ff1_w, ff1_b, ff2_w, ff2_b, cl_w, cl_b):
    del interactions
    seq, n_nodes, _ = user_profiles.shape
    n_edges = edge_index.shape[1]
    gdim = gcn_w.shape[1]
    bn = 2000
    n_pad = -(-n_nodes // (NS * 128)) * (NS * 128)

    src = edge_index[0].astype(jnp.int32)
    dst = edge_index[1].astype(jnp.int32)

    degp = _make_deg_kernel(n_pad, n_edges)(dst)
    xw = _run_gru(user_profiles, W_ih, W_hh, b_ih.reshape(1, -1),
                  b_hh.reshape(1, -1), gcn_w, bn)
    y = _run_scale(xw, degp, bn)
    accp = _make_edge_kernel(n_pad, n_edges, gdim)(y, src, dst)
    max_depth, clusters = _run_heads(
        accp, y, degp, gcn_b.reshape(1, -1), ff1_w, ff1_b.reshape(1, -1),
        ff2_w, ff2_b.reshape(1, -1), cl_w, cl_b.reshape(1, -1), bn)
    return (max_depth, clusters)
